# baseline (speedup 1.0000x reference)
import functools

import jax
import jax.numpy as jnp
from jax import lax
from jax.experimental import pallas as pl
from jax.experimental.pallas import tpu as pltpu
from jax.experimental.pallas import tpu_sc as plsc

B, H, N = 32, 8, 8192
ROWS = B * H
L = 16
CHUNKS = N // L
UNROLL = 4
NUM_WORKERS = 32
ROWS_PER_W = ROWS // NUM_WORKERS

_mesh = plsc.VectorSubcoreMesh(core_axis_name="c", subcore_axis_name="s")


@functools.partial(
    pl.kernel,
    out_type=jax.ShapeDtypeStruct((NUM_WORKERS, L), jnp.int32),
    mesh=_mesh,
    scratch_types=[
        pltpu.VMEM((2, N), jnp.float32),
        pltpu.VMEM((L,), jnp.int32),
        pltpu.SemaphoreType.DMA,
        pltpu.SemaphoreType.DMA,
    ],
    compiler_params=pltpu.CompilerParams(needs_layout_passes=False),
)
def _argmin_sc(x_hbm, out_hbm, buf, res_v, sem0, sem1):
    wid = lax.axis_index("s") * 2 + lax.axis_index("c")
    base = wid * ROWS_PER_W
    lane = jnp.arange(L, dtype=jnp.int32)
    sems = (sem0, sem1)

    cps = [None, None]
    cps[0] = pltpu.async_copy(x_hbm.at[base], buf.at[0], sems[0])

    res = jnp.zeros((L,), jnp.int32)
    for r in range(ROWS_PER_W):
        slot = r % 2
        if r + 1 < ROWS_PER_W:
            nxt = (r + 1) % 2
            cps[nxt] = pltpu.async_copy(
                x_hbm.at[base + r + 1], buf.at[nxt], sems[nxt]
            )
        cps[slot].wait()

        def step(j, carry, slot=slot):
            best, bchunk = carry
            for u in range(UNROLL):
                c = j * UNROLL + u
                v = buf[slot, pl.ds(c * L, L)]
                m = v < best
                best = jnp.where(m, v, best)
                bchunk = jnp.where(m, c, bchunk)
            return best, bchunk

        best, bchunk = lax.fori_loop(
            0,
            CHUNKS // UNROLL,
            step,
            (jnp.full((L,), jnp.inf, jnp.float32), jnp.zeros((L,), jnp.int32)),
        )

        minval = jnp.min(best)
        cand = jnp.where(best == minval, bchunk * L + lane, jnp.int32(N))
        row_idx = jnp.min(cand)
        res = jnp.where(lane == r, row_idx, res)

    res_v[...] = res
    pltpu.sync_copy(res_v, out_hbm.at[wid])


def kernel(x):
    out = _argmin_sc(x.reshape(ROWS, N))
    return out[:, :ROWS_PER_W].reshape(B, H)


# speedup vs baseline: 1.0025x; 1.0025x over previous
import functools

import jax
import jax.numpy as jnp
from jax import lax
from jax.experimental import pallas as pl
from jax.experimental.pallas import tpu as pltpu
from jax.experimental.pallas import tpu_sc as plsc

B, H, N = 32, 8, 8192
ROWS = B * H
L = 16
CHUNKS = N // L
UNROLL = 4
NUM_WORKERS = 32
ROWS_PER_W = ROWS // NUM_WORKERS

_mesh = plsc.VectorSubcoreMesh(core_axis_name="c", subcore_axis_name="s")


@functools.partial(
    pl.kernel,
    out_type=jax.ShapeDtypeStruct((NUM_WORKERS, L), jnp.int32),
    mesh=_mesh,
    scratch_types=[
        pltpu.VMEM((ROWS_PER_W, N), jnp.float32),
        pltpu.VMEM((L,), jnp.int32),
        pltpu.SemaphoreType.DMA,
        pltpu.SemaphoreType.DMA,
    ],
    compiler_params=pltpu.CompilerParams(needs_layout_passes=False),
)
def _argmin_sc(x_hbm, out_hbm, buf, res_v, sem0, sem1):
    wid = lax.axis_index("s") * 2 + lax.axis_index("c")
    base = wid * ROWS_PER_W
    lane = jnp.arange(L, dtype=jnp.int32)

    pltpu.async_copy(
        x_hbm.at[pl.ds(base, ROWS_PER_W)], buf, sem0
    ).wait()

    res = jnp.zeros((L,), jnp.int32)
    for r in range(ROWS_PER_W):
        slot = r

        def step(j, carry, slot=slot):
            best, bchunk = carry
            for u in range(UNROLL):
                c = j * UNROLL + u
                v = buf[slot, pl.ds(c * L, L)]
                m = v < best
                best = jnp.where(m, v, best)
                bchunk = jnp.where(m, c, bchunk)
            return best, bchunk

        best, bchunk = lax.fori_loop(
            0,
            CHUNKS // UNROLL,
            step,
            (jnp.full((L,), jnp.inf, jnp.float32), jnp.zeros((L,), jnp.int32)),
        )

        minval = jnp.min(best)
        cand = jnp.where(best == minval, bchunk * L + lane, jnp.int32(N))
        row_idx = jnp.min(cand)
        res = jnp.where(lane == r, row_idx, res)

    res_v[...] = res
    pltpu.sync_copy(res_v, out_hbm.at[wid])


def kernel(x):
    out = _argmin_sc(x.reshape(ROWS, N))
    return out[:, :ROWS_PER_W].reshape(B, H)
